# baseline (speedup 1.0000x reference)
import functools
import jax
import jax.numpy as jnp
from jax import lax
from jax.experimental import pallas as pl
from jax.experimental.pallas import tpu as pltpu
from jax.experimental.pallas import tpu_sc as plsc

MAX_LEN = 8192
EMBED_DIM = 768
BATCH = 4

NC, NS = 2, 16
NW = NC * NS
S_PER_W = MAX_LEN // NW
R = 32
CHUNKS = S_PER_W // R
CE = R * EMBED_DIM
VREGS = CE // 16

_mesh = plsc.VectorSubcoreMesh(core_axis_name="c", subcore_axis_name="s")


@functools.partial(
    pl.kernel,
    out_type=jax.ShapeDtypeStruct((BATCH * MAX_LEN * EMBED_DIM,), jnp.float32),
    mesh=_mesh,
    scratch_types=[
        pltpu.VMEM((CE,), jnp.float32),
        pltpu.VMEM((CE,), jnp.float32),
    ],
)
def _sc_add(x_hbm, t_hbm, out_hbm, tbuf, xbuf):
    wid = lax.axis_index("s") * NC + lax.axis_index("c")
    s0 = wid * S_PER_W
    for c in range(CHUNKS):
        toff = (s0 + c * R) * EMBED_DIM
        pltpu.sync_copy(t_hbm.at[pl.ds(toff, CE)], tbuf)
        for b in range(BATCH):
            xoff = (b * MAX_LEN + s0 + c * R) * EMBED_DIM
            pltpu.sync_copy(x_hbm.at[pl.ds(xoff, CE)], xbuf)

            def body(i, _):
                sl = pl.ds(pl.multiple_of(i * 16, 16), 16)
                xbuf[sl] = xbuf[sl] + tbuf[sl]
                return 0

            lax.fori_loop(0, VREGS, body, 0, unroll=8)
            pltpu.sync_copy(xbuf, out_hbm.at[pl.ds(xoff, CE)])


def kernel(x, table):
    out = _sc_add(x.reshape(-1), table.reshape(-1))
    return out.reshape(BATCH, MAX_LEN, EMBED_DIM)
